# baseline (speedup 1.0000x reference)
import jax
import jax.numpy as jnp
from jax.experimental import pallas as pl
from jax.experimental.pallas import tpu as pltpu

_TEMPERATURE = 0.7
_N_ITERS = 8
_EPS = 1e-6


def _uniform(b, num_tokens, n):
    return jax.random.uniform(jax.random.key(42), (b, num_tokens, n),
                              minval=_EPS, maxval=1.0 - _EPS)


def _scores_kernel(x_ref, rt_ref, base_ref):
    prod = x_ref[0] * rt_ref[...]
    s = jnp.sum(prod, axis=1, keepdims=True)
    base_ref[0] = s / _TEMPERATURE


def _sinkhorn_kernel(u_ref, base_ref, ones_ref, idx_ref, t_ref):
    t = -jnp.log(-jnp.log(u_ref[0])) + base_ref[0]
    t_ref[...] = t
    m0 = jnp.max(t, axis=1, keepdims=True)
    for _ in range(_N_ITERS):
        s0 = jnp.sum(jnp.exp(t_ref[...] - m0), axis=1, keepdims=True)
        t = t_ref[...] - (jnp.log(s0) + m0)
        t_ref[...] = t
        m1 = jnp.max(t, axis=0, keepdims=True)
        s1 = jnp.sum(jnp.exp(t_ref[...] - m1), axis=0, keepdims=True)
        t = t_ref[...] - (jnp.log(s1) + m1)
        t_ref[...] = t
        m0 = jnp.max(t, axis=1, keepdims=True)
    t = t_ref[...]
    iota = jax.lax.broadcasted_iota(jnp.int32, t.shape, 1)
    idx = jnp.min(jnp.where(t == m0, iota, t.shape[1]), axis=1, keepdims=True)
    idx_ref[0] = idx
    ones_ref[0] = jnp.ones_like(m0)


def kernel(x, routing_token, num_tokens):
    b, n, d = x.shape
    nt = routing_token.shape[0]
    del num_tokens

    base = pl.pallas_call(
        _scores_kernel,
        grid=(b,),
        in_specs=[
            pl.BlockSpec((1, n, d), lambda i: (i, 0, 0)),
            pl.BlockSpec((1, d), lambda i: (0, 0)),
        ],
        out_specs=pl.BlockSpec((1, n, 1), lambda i: (i, 0, 0)),
        out_shape=jax.ShapeDtypeStruct((b, n, 1), jnp.float32),
    )(x, routing_token.reshape(1, d))

    ones, idx = pl.pallas_call(
        _sinkhorn_kernel,
        grid=(b,),
        in_specs=[
            pl.BlockSpec((1, nt, n), lambda i: (i, 0, 0)),
            pl.BlockSpec((1, 1, n), lambda i: (i, 0, 0)),
        ],
        out_specs=[
            pl.BlockSpec((1, nt, 1), lambda i: (i, 0, 0)),
            pl.BlockSpec((1, nt, 1), lambda i: (i, 0, 0)),
        ],
        out_shape=[
            jax.ShapeDtypeStruct((b, nt, 1), jnp.float32),
            jax.ShapeDtypeStruct((b, nt, 1), jnp.int32),
        ],
        scratch_shapes=[pltpu.VMEM((nt, n), jnp.float32)],
    )(_uniform(b, nt, n), base.reshape(b, 1, n))

    return ones.reshape(b, nt), idx.reshape(b, nt)


# speedup vs baseline: 1.1570x; 1.1570x over previous
import jax
import jax.numpy as jnp
from jax.experimental import pallas as pl
from jax.experimental.pallas import tpu as pltpu

_TEMPERATURE = 0.7
_N_ITERS = 8
_EPS = 1e-6


def _gumbel(b, num_tokens, n):
    u = jax.random.uniform(jax.random.key(42), (b, num_tokens, n),
                           minval=_EPS, maxval=1.0 - _EPS)
    return -jnp.log(-jnp.log(u))


def _scores_kernel(x_ref, rt_ref, base_ref):
    prod = x_ref[0] * rt_ref[...]
    s = jnp.sum(prod, axis=1, keepdims=True)
    base_ref[0] = s / _TEMPERATURE


def _sinkhorn_kernel(g_ref, base_ref, ones_ref, idx_ref, t_ref):
    t = g_ref[0] + base_ref[0]
    t_ref[...] = t
    m0 = jnp.max(t, axis=1, keepdims=True)
    for _ in range(_N_ITERS):
        s0 = jnp.sum(jnp.exp(t_ref[...] - m0), axis=1, keepdims=True)
        t = t_ref[...] - (jnp.log(s0) + m0)
        t_ref[...] = t
        m1 = jnp.max(t, axis=0, keepdims=True)
        s1 = jnp.sum(jnp.exp(t_ref[...] - m1), axis=0, keepdims=True)
        t = t_ref[...] - (jnp.log(s1) + m1)
        t_ref[...] = t
        m0 = jnp.max(t, axis=1, keepdims=True)
    t = t_ref[...]
    iota = jax.lax.broadcasted_iota(jnp.int32, t.shape, 1)
    idx = jnp.min(jnp.where(t == m0, iota, t.shape[1]), axis=1, keepdims=True)
    idx_ref[0] = idx
    ones_ref[0] = jnp.ones_like(m0)


def kernel(x, routing_token, num_tokens):
    b, n, d = x.shape
    nt = routing_token.shape[0]
    del num_tokens

    base = pl.pallas_call(
        _scores_kernel,
        grid=(b,),
        in_specs=[
            pl.BlockSpec((1, n, d), lambda i: (i, 0, 0)),
            pl.BlockSpec((1, d), lambda i: (0, 0)),
        ],
        out_specs=pl.BlockSpec((1, n, 1), lambda i: (i, 0, 0)),
        out_shape=jax.ShapeDtypeStruct((b, n, 1), jnp.float32),
    )(x, routing_token.reshape(1, d))

    ones, idx = pl.pallas_call(
        _sinkhorn_kernel,
        grid=(b,),
        in_specs=[
            pl.BlockSpec((1, nt, n), lambda i: (i, 0, 0)),
            pl.BlockSpec((1, 1, n), lambda i: (i, 0, 0)),
        ],
        out_specs=[
            pl.BlockSpec((1, nt, 1), lambda i: (i, 0, 0)),
            pl.BlockSpec((1, nt, 1), lambda i: (i, 0, 0)),
        ],
        out_shape=[
            jax.ShapeDtypeStruct((b, nt, 1), jnp.float32),
            jax.ShapeDtypeStruct((b, nt, 1), jnp.int32),
        ],
        scratch_shapes=[pltpu.VMEM((nt, n), jnp.float32)],
    )(_gumbel(b, nt, n), base.reshape(b, 1, n))

    return ones.reshape(b, nt), idx.reshape(b, nt)


# speedup vs baseline: 1.2129x; 1.0483x over previous
import jax
import jax.numpy as jnp
from jax.experimental import pallas as pl
from jax.experimental.pallas import tpu as pltpu

_TEMPERATURE = 0.7
_N_ITERS = 8
_EPS = 1e-6

def _gumbel_t(b, num_tokens, n):
    u = jax.random.uniform(jax.random.key(42), (b, num_tokens, n),
                           minval=_EPS, maxval=1.0 - _EPS)
    return jnp.swapaxes(-jnp.log(-jnp.log(u)), 1, 2)


def _router_kernel(x_ref, rt_ref, g_ref, ones_ref, idx_ref, t_ref):
    s = jnp.sum(x_ref[0] * rt_ref[...], axis=1, keepdims=True)
    u = g_ref[0] + s / _TEMPERATURE
    t_ref[...] = u
    m0 = jnp.max(u, axis=0, keepdims=True)
    for _ in range(_N_ITERS):
        s0 = jnp.sum(jnp.exp(t_ref[...] - m0), axis=0, keepdims=True)
        u = t_ref[...] - (jnp.log(s0) + m0)
        t_ref[...] = u
        m1 = jnp.max(u, axis=1, keepdims=True)
        s1 = jnp.sum(jnp.exp(t_ref[...] - m1), axis=1, keepdims=True)
        u = t_ref[...] - (jnp.log(s1) + m1)
        t_ref[...] = u
        m0 = jnp.max(u, axis=0, keepdims=True)
    t = t_ref[...]
    iota = jax.lax.broadcasted_iota(jnp.int32, t.shape, 0)
    idx = jnp.min(jnp.where(t == m0, iota, t.shape[0]), axis=0, keepdims=True)
    idx_ref[0] = idx
    ones_ref[0] = jnp.ones_like(m0)


def kernel(x, routing_token, num_tokens):
    b, n, d = x.shape
    nt = routing_token.shape[0]
    del num_tokens

    ones, idx = pl.pallas_call(
        _router_kernel,
        grid=(b,),
        in_specs=[
            pl.BlockSpec((1, n, d), lambda i: (i, 0, 0)),
            pl.BlockSpec((1, d), lambda i: (0, 0)),
            pl.BlockSpec((1, n, nt), lambda i: (i, 0, 0)),
        ],
        out_specs=[
            pl.BlockSpec((1, 1, nt), lambda i: (i, 0, 0)),
            pl.BlockSpec((1, 1, nt), lambda i: (i, 0, 0)),
        ],
        out_shape=[
            jax.ShapeDtypeStruct((b, 1, nt), jnp.float32),
            jax.ShapeDtypeStruct((b, 1, nt), jnp.int32),
        ],
        scratch_shapes=[pltpu.VMEM((n, nt), jnp.float32)],
    )(x, routing_token.reshape(1, d), _gumbel_t(b, nt, n))

    return ones.reshape(b, nt), idx.reshape(b, nt)
